# baseline (speedup 1.0000x reference)
import jax
import jax.numpy as jnp
from jax import lax
from jax.experimental import pallas as pl
from jax.experimental.pallas import tpu as pltpu
from jax.experimental.pallas import tpu_sc as plsc

FEATS = 136
K = 5000
N_NODES = 100000
BM = 2048
N_PAD = 100352

NW = 16
CHUNK = N_PAD // NW
VECS = CHUNK // 16
COMP = 5120
TRASH = COMP
CARR = 5376
INITW = CARR // NW
GROWS = COMP // NW
NEG_INF = float("-inf")


def _matvec_body(x_ref, s_ref, o_ref):
    i = pl.program_id(0)
    raw = jnp.dot(x_ref[...], s_ref[...], preferred_element_type=jnp.float32)
    rid = lax.broadcasted_iota(jnp.int32, (BM, 1), 0) + i * BM
    o_ref[...] = jnp.where(rid < N_NODES, raw, NEG_INF)


def _matvec(node_embs, scorer):
    return pl.pallas_call(
        _matvec_body,
        grid=(N_PAD // BM,),
        in_specs=[
            pl.BlockSpec((BM, FEATS), lambda i: (i, 0)),
            pl.BlockSpec((FEATS, 1), lambda i: (0, 0)),
        ],
        out_specs=pl.BlockSpec((BM, 1), lambda i: (i, 0)),
        out_shape=jax.ShapeDtypeStruct((N_PAD, 1), jnp.float32),
    )(node_embs, scorer)


def _bci(x):
    return lax.broadcast_in_dim(jnp.asarray(x, jnp.int32), (16,), ())


def _bcu(x):
    return lax.broadcast_in_dim(jnp.asarray(x, jnp.uint32), (16,), ())


def _select_body(scores_hbm, sco_out, sidx_out,
                 chunk_v, keys_v, hist_v, gridbuf_v,
                 initf_v, initi_v, locgf_v, locgi_v, locef_v, locei_v,
                 grid_sh, semf, semi):
    wid = lax.axis_index("s")
    lane = lax.broadcasted_iota(jnp.int32, (16,), 0)
    cbase = wid * CHUNK

    pltpu.sync_copy(scores_hbm.at[pl.ds(pl.multiple_of(cbase, 64), CHUNK)],
                    chunk_v)

    def _mk_keys(i, _):
        s = chunk_v[pl.ds(i * 16, 16)] + 0.0
        b = lax.bitcast_convert_type(s, jnp.int32)
        bu = lax.bitcast_convert_type(b, jnp.uint32)
        key = jnp.where(b < 0, ~bu, bu | jnp.uint32(0x80000000))
        keys_v[pl.ds(i * 16, 16)] = key
        return 0

    lax.fori_loop(0, VECS, _mk_keys, 0)

    prefix = jnp.uint32(0)
    k_rem = jnp.int32(K)
    g_acc = jnp.int32(0)
    ones16 = jnp.ones((16,), jnp.int32)
    for r in (3, 2, 1, 0):
        for j in range(16):
            hist_v[pl.ds(j * 16, 16)] = jnp.zeros((16,), jnp.int32)

        shift_d = jnp.uint32(8 * r)
        shift_e = jnp.uint32(8 * (r + 1))

        def _hist(i, _, r=r, shift_d=shift_d, shift_e=shift_e, prefix=prefix):
            kk = keys_v[pl.ds(i * 16, 16)]
            digit = ((kk >> shift_d) & jnp.uint32(0xFF)).astype(jnp.int32)
            if r == 3:
                elig = lane < 16
            else:
                elig = (kk >> shift_e) == _bcu(prefix)
            plsc.addupdate_scatter(hist_v, [digit], ones16, mask=elig)
            return 0

        lax.fori_loop(0, VECS, _hist, 0)

        pltpu.sync_copy(hist_v, grid_sh.at[wid])
        plsc.subcore_barrier()
        pltpu.sync_copy(grid_sh, gridbuf_v)
        plsc.subcore_barrier()

        gh = []
        for c in range(16):
            acc = jnp.zeros((16,), jnp.int32)
            for t in range(16):
                acc = acc + gridbuf_v[t, pl.ds(c * 16, 16)]
            gh.append(acc)

        carry = jnp.int32(0)
        d_star = jnp.int32(0)
        krem_new = jnp.int32(0)
        for c in reversed(range(16)):
            v = gh[c]
            sfx_incl = lax.rev(plsc.cumsum(lax.rev(v, (0,))), (0,))
            sfx_excl = sfx_incl - v + _bci(carry)
            cond = (_bci(k_rem) > sfx_excl) & (_bci(k_rem) <= sfx_excl + v)
            d_star = d_star + jnp.sum(jnp.where(cond, c * 16 + lane, 0))
            krem_new = krem_new + jnp.sum(
                jnp.where(cond, _bci(k_rem) - sfx_excl, 0))
            carry = carry + jnp.sum(v)
        g_acc = g_acc + (k_rem - krem_new)
        k_rem = krem_new
        prefix = (prefix << jnp.uint32(8)) | d_star.astype(jnp.uint32)

    t_key = prefix
    g_tot = g_acc
    m_eq = k_rem

    def _cnt(i, cacc):
        ga, ea = cacc
        kk = keys_v[pl.ds(i * 16, 16)]
        ga = ga + jnp.where(kk > _bcu(t_key), 1, 0)
        ea = ea + jnp.where(kk == _bcu(t_key), 1, 0)
        return (ga, ea)

    gacc, eacc = lax.fori_loop(
        0, VECS, _cnt,
        (jnp.zeros((16,), jnp.int32), jnp.zeros((16,), jnp.int32)))
    c_gt = jnp.sum(gacc)
    c_eq = jnp.sum(eacc)

    hist_v[pl.ds(0, 16)] = jnp.where(lane == 0, _bci(c_gt),
                                     jnp.where(lane == 1, _bci(c_eq), 0))
    for j in range(1, 16):
        hist_v[pl.ds(j * 16, 16)] = jnp.zeros((16,), jnp.int32)
    plsc.subcore_barrier()
    pltpu.sync_copy(hist_v, grid_sh.at[wid])
    plsc.subcore_barrier()
    pltpu.sync_copy(grid_sh, gridbuf_v)

    gt_base = jnp.int32(0)
    eq_before = jnp.int32(0)
    for t in range(16):
        row = gridbuf_v[t, pl.ds(0, 16)]
        g_t = jnp.sum(jnp.where(lane == 0, row, 0))
        e_t = jnp.sum(jnp.where(lane == 1, row, 0))
        before = jnp.int32(t) < wid
        gt_base = gt_base + jnp.where(before, g_t, 0)
        eq_before = eq_before + jnp.where(before, e_t, 0)

    for j in range(INITW // 16):
        initf_v[pl.ds(j * 16, 16)] = jnp.full((16,), NEG_INF, jnp.float32)
        initi_v[pl.ds(j * 16, 16)] = jnp.zeros((16,), jnp.int32)
    ibase = pl.multiple_of(wid * INITW, 16)
    pltpu.sync_copy(initf_v, sco_out.at[pl.ds(ibase, INITW)])
    pltpu.sync_copy(initi_v, sidx_out.at[pl.ds(ibase, INITW)])
    plsc.subcore_barrier()

    def _compact(i, rc):
        r_gt, r_eq = rc
        kk = keys_v[pl.ds(i * 16, 16)]
        sv = chunk_v[pl.ds(i * 16, 16)]
        idxv = _bci(cbase + i * 16) + lane
        m_gt = kk > _bcu(t_key)
        m_e = kk == _bcu(t_key)
        x_gt = jnp.where(m_gt, 1, 0)
        x_eq = jnp.where(m_e, 1, 0)
        excl_gt = plsc.cumsum(x_gt) - x_gt
        excl_eq = plsc.cumsum(x_eq) - x_eq
        pg = _bci(r_gt) + excl_gt
        pe = _bci(r_eq) + excl_eq
        plsc.store_scatter(locgf_v, [pg], sv, mask=m_gt)
        plsc.store_scatter(locgi_v, [pg], idxv, mask=m_gt)
        plsc.store_scatter(locef_v, [pe], sv, mask=m_e)
        plsc.store_scatter(locei_v, [pe], idxv, mask=m_e)
        return (r_gt + jnp.sum(x_gt), r_eq + jnp.sum(x_eq))

    c_gt2, _ = lax.fori_loop(0, VECS, _compact, (jnp.int32(0), jnp.int32(0)))

    def _emit_gt(t, _):
        off = t * 16
        valid = (_bci(off) + lane) < _bci(c_gt)
        dst = jnp.where(valid, _bci(gt_base + off) + lane, TRASH + lane)
        cf = pltpu.async_copy(locgf_v.at[pl.ds(pl.multiple_of(off, 16), 16)],
                              sco_out.at[dst], semf)
        ci = pltpu.async_copy(locgi_v.at[pl.ds(pl.multiple_of(off, 16), 16)],
                              sidx_out.at[dst], semi)
        cf.wait()
        ci.wait()
        return 0

    lax.fori_loop(0, (c_gt + 15) // 16, _emit_gt, 0)

    n_eq = jnp.maximum(jnp.int32(0), jnp.minimum(c_eq, m_eq - eq_before))
    ebase = g_tot + eq_before

    def _emit_eq(t, _):
        off = t * 16
        valid = (_bci(off) + lane) < _bci(n_eq)
        dst = jnp.where(valid, _bci(ebase + off) + lane, TRASH + lane)
        cf = pltpu.async_copy(locef_v.at[pl.ds(pl.multiple_of(off, 16), 16)],
                              sco_out.at[dst], semf)
        ci = pltpu.async_copy(locei_v.at[pl.ds(pl.multiple_of(off, 16), 16)],
                              sidx_out.at[dst], semi)
        cf.wait()
        ci.wait()
        return 0

    lax.fori_loop(0, (n_eq + 15) // 16, _emit_eq, 0)


def _sc_select(scores_padded):
    mesh = plsc.VectorSubcoreMesh(core_axis_name="c", subcore_axis_name="s",
                                  num_cores=1, num_subcores=16)
    fn = pl.kernel(
        _select_body,
        out_type=(jax.ShapeDtypeStruct((CARR,), jnp.float32),
                  jax.ShapeDtypeStruct((CARR,), jnp.int32)),
        mesh=mesh,
        scratch_types=[
            pltpu.VMEM((CHUNK,), jnp.float32),
            pltpu.VMEM((CHUNK,), jnp.uint32),
            pltpu.VMEM((256,), jnp.int32),
            pltpu.VMEM((16, 256), jnp.int32),
            pltpu.VMEM((INITW,), jnp.float32),
            pltpu.VMEM((INITW,), jnp.int32),
            pltpu.VMEM((CHUNK,), jnp.float32),
            pltpu.VMEM((CHUNK,), jnp.int32),
            pltpu.VMEM((CHUNK,), jnp.float32),
            pltpu.VMEM((CHUNK,), jnp.int32),
            pltpu.VMEM_SHARED((16, 256), jnp.int32),
            pltpu.SemaphoreType.DMA,
            pltpu.SemaphoreType.DMA,
        ],
        compiler_params=pltpu.CompilerParams(needs_layout_passes=False),
    )
    return fn(scores_padded)


def _rank_body(sa_ref, sb_ref, rank_ref):
    j = pl.program_id(0)
    sa = sa_ref[...]
    sb = sb_ref[...].reshape(1, 128)
    pa = lax.broadcasted_iota(jnp.int32, (COMP, 1), 0)
    pb = lax.broadcasted_iota(jnp.int32, (1, 128), 1) + j * 128
    beats = (sb > sa) | ((sb == sa) & (pb < pa))
    cnt = jnp.sum(beats.astype(jnp.float32), axis=1, keepdims=True)

    @pl.when(j == 0)
    def _():
        rank_ref[...] = cnt

    @pl.when(j > 0)
    def _():
        rank_ref[...] = rank_ref[...] + cnt


def _rank(cs_col, cs_rows):
    return pl.pallas_call(
        _rank_body,
        grid=(COMP // 128,),
        in_specs=[
            pl.BlockSpec((COMP, 1), lambda j: (0, 0)),
            pl.BlockSpec((1, 1, 128), lambda j: (j, 0, 0)),
        ],
        out_specs=pl.BlockSpec((COMP, 1), lambda j: (0, 0)),
        out_shape=jax.ShapeDtypeStruct((COMP, 1), jnp.float32),
    )(cs_col, cs_rows)


def _permute_body(rank_ref, ci_ref, cs_ref, oi_ref, os_ref):
    j = pl.program_id(0)
    rk = rank_ref[...]
    rb = (lax.broadcasted_iota(jnp.int32, (1, 128), 1) + j * 128)
    oh = (rk == rb.astype(jnp.float32)).astype(jnp.float32)
    cs_fin = cs_ref[...]
    cs_fin = jnp.where(cs_fin == NEG_INF, 0.0, cs_fin)
    dn = (((0,), (0,)), ((), ()))
    oi_ref[...] = lax.dot_general(oh, ci_ref[...], dn,
                                  precision=lax.Precision.HIGHEST,
                                  preferred_element_type=jnp.float32)
    os_ref[...] = lax.dot_general(oh, cs_fin, dn,
                                  precision=lax.Precision.HIGHEST,
                                  preferred_element_type=jnp.float32)


def _permute(rank, ci_col, cs_col):
    return pl.pallas_call(
        _permute_body,
        grid=(COMP // 128,),
        in_specs=[
            pl.BlockSpec((COMP, 1), lambda j: (0, 0)),
            pl.BlockSpec((COMP, 1), lambda j: (0, 0)),
            pl.BlockSpec((COMP, 1), lambda j: (0, 0)),
        ],
        out_specs=[
            pl.BlockSpec((128, 1), lambda j: (j, 0)),
            pl.BlockSpec((128, 1), lambda j: (j, 0)),
        ],
        out_shape=[
            jax.ShapeDtypeStruct((COMP, 1), jnp.float32),
            jax.ShapeDtypeStruct((COMP, 1), jnp.float32),
        ],
    )(rank, ci_col, cs_col)


def _gather_body(embs_hbm, sidx_hbm, out_hbm, idxv_v, rows_v, sem):
    wid = lax.axis_index("s")
    base = pl.multiple_of(wid * GROWS, 16)
    pltpu.sync_copy(sidx_hbm.at[pl.ds(base, GROWS)], idxv_v)
    pltpu.async_copy(embs_hbm.at[idxv_v], rows_v, sem).wait()
    pltpu.sync_copy(rows_v, out_hbm.at[pl.ds(base, GROWS)])


def _sc_gather(node_embs, sidx):
    mesh = plsc.VectorSubcoreMesh(core_axis_name="c", subcore_axis_name="s",
                                  num_cores=1, num_subcores=16)
    fn = pl.kernel(
        _gather_body,
        out_type=jax.ShapeDtypeStruct((COMP, FEATS), jnp.float32),
        mesh=mesh,
        scratch_types=[
            pltpu.VMEM((GROWS,), jnp.int32),
            pltpu.VMEM((GROWS, FEATS), jnp.float32),
            pltpu.SemaphoreType.DMA,
        ],
        compiler_params=pltpu.CompilerParams(needs_layout_passes=False,
                                             use_tc_tiling_on_sc=False),
    )
    return fn(node_embs, sidx)


def _finish_body(g_ref, s_ref, o_ref):
    gate = jnp.tanh(s_ref[...])
    scaled = g_ref[...] * gate
    e0 = lax.broadcasted_iota(jnp.int32, (128, 128), 0)
    e1 = lax.broadcasted_iota(jnp.int32, (128, 128), 1)
    eye = (e0 == e1).astype(jnp.float32)
    dn = (((0,), (0,)), ((), ()))
    o_ref[...] = lax.dot_general(scaled, eye, dn,
                                 precision=lax.Precision.HIGHEST,
                                 preferred_element_type=jnp.float32)


def _finish(gath, sscore):
    return pl.pallas_call(
        _finish_body,
        grid=(COMP // 128,),
        in_specs=[
            pl.BlockSpec((128, FEATS), lambda j: (j, 0)),
            pl.BlockSpec((128, 1), lambda j: (j, 0)),
        ],
        out_specs=pl.BlockSpec((FEATS, 128), lambda j: (0, j)),
        out_shape=jax.ShapeDtypeStruct((FEATS, COMP), jnp.float32),
    )(gath, sscore)


_DEBUG_XLA_TAIL = False
def kernel(node_embs, scorer):
    raw = _matvec(node_embs, scorer)
    flat = (raw / jnp.linalg.norm(scorer)).reshape(-1)
    sco_c, sidx_c = _sc_select(flat)
    if _DEBUG_XLA_TAIL:
        cs_d = sco_c[:COMP]
        ci_d = sidx_c[:COMP]
        perm = jnp.lexsort((jnp.arange(COMP), -cs_d))
        sidx_d = ci_d[perm][:K]
        vals_d = cs_d[perm][:K]
        gath_d = jnp.take(node_embs, sidx_d, axis=0)
        return (gath_d * jnp.tanh(vals_d)[:, None]).T
    cs = sco_c[:COMP].reshape(COMP, 1)
    ci = sidx_c[:COMP].astype(jnp.float32).reshape(COMP, 1)
    rank = _rank(cs, cs.reshape(COMP // 128, 1, 128))
    sif, ssf = _permute(rank, ci, cs)
    sidx = (sif.reshape(-1) + 0.5).astype(jnp.int32)
    gath = _sc_gather(node_embs, sidx)
    outt = _finish(gath, ssf)
    return outt[:, :K]
